# Initial kernel scaffold; baseline (speedup 1.0000x reference)
import functools

import jax
import jax.numpy as jnp
from jax import lax
from jax.experimental import pallas as pl
from jax.experimental.pallas import tpu as pltpu
from jax.experimental.pallas import tpu_sc as plsc

_VOCAB = 1000000
_D = 64
_ODIM = 128
_B = 16384
_CTX = 50

_NC = 2
_NS = 16
_NW = _NC * _NS
_BPW = _B // _NW
_CB = 8
_NCHUNK = _BPW // _CB
_IDX_PER_CHUNK = _CB * _CTX
_GATHER_W = 80
_NGATHER = _IDX_PER_CHUNK // _GATHER_W


def _sc_pool(idx2d, embed):
    mesh = plsc.VectorSubcoreMesh(core_axis_name="c", subcore_axis_name="s")

    @functools.partial(
        pl.kernel,
        mesh=mesh,
        out_type=jax.ShapeDtypeStruct((_B, _D), jnp.float32),
        scratch_types=[
            pltpu.VMEM((_NGATHER, _GATHER_W), jnp.int32),
            pltpu.VMEM((_IDX_PER_CHUNK, _D), jnp.float32),
            pltpu.VMEM((_CB, _D), jnp.float32),
            pltpu.SemaphoreType.DMA,
        ],
    )
    def k(idx_hbm, table_hbm, out_hbm, idx_v, rows_v, acc_v, sem):
        wid = lax.axis_index("s") * _NC + lax.axis_index("c")
        row0 = wid * (_BPW * _CTX // _GATHER_W)

        def chunk_body(i, carry):
            pltpu.sync_copy(idx_hbm.at[pl.ds(row0 + i * _NGATHER, _NGATHER)], idx_v)
            cps = [
                pltpu.async_copy(
                    table_hbm.at[idx_v.at[s]],
                    rows_v.at[pl.ds(s * _GATHER_W, _GATHER_W)],
                    sem,
                )
                for s in range(_NGATHER)
            ]
            for cp in cps:
                cp.wait()
            for bb in range(_CB):
                base_r = bb * _CTX
                zeros = jnp.zeros((16,), jnp.float32)
                def ctx_body(c, acc, base_r=base_r):
                    r = base_r + c * 5
                    a0, a1, a2, a3 = acc
                    for u in range(5):
                        a0 = a0 + rows_v[r + u, pl.ds(0, 16)]
                        a1 = a1 + rows_v[r + u, pl.ds(16, 16)]
                        a2 = a2 + rows_v[r + u, pl.ds(32, 16)]
                        a3 = a3 + rows_v[r + u, pl.ds(48, 16)]
                    return (a0, a1, a2, a3)
                a0, a1, a2, a3 = lax.fori_loop(
                    0, _CTX // 5, ctx_body, (zeros, zeros, zeros, zeros))
                acc_v[bb, pl.ds(0, 16)] = a0
                acc_v[bb, pl.ds(16, 16)] = a1
                acc_v[bb, pl.ds(32, 16)] = a2
                acc_v[bb, pl.ds(48, 16)] = a3
            b0 = wid * _BPW + i * _CB
            pltpu.sync_copy(acc_v, out_hbm.at[pl.ds(b0, _CB)])
            return carry

        lax.fori_loop(0, _NCHUNK, chunk_body, 0)

    return k(idx2d, embed)


def _tc_linear(pooled, W, b2d):
    BB = 2048

    def body(x_ref, w_ref, b_ref, o_ref):
        o_ref[...] = lax.dot_general(
            x_ref[...], w_ref[...], (((1,), (1,)), ((), ())),
            preferred_element_type=jnp.float32,
        ) + b_ref[...]

    return pl.pallas_call(
        body,
        grid=(_B // BB,),
        in_specs=[
            pl.BlockSpec((BB, _D), lambda i: (i, 0)),
            pl.BlockSpec((_ODIM, _D), lambda i: (0, 0)),
            pl.BlockSpec((1, _ODIM), lambda i: (0, 0)),
        ],
        out_specs=pl.BlockSpec((BB, _ODIM), lambda i: (i, 0)),
        out_shape=jax.ShapeDtypeStruct((_B, _ODIM), jnp.float32),
    )(pooled, W, b2d)


def kernel(inputs, embed, W, b):
    idx2d = inputs.astype(jnp.int32).reshape(_B * _CTX // _GATHER_W, _GATHER_W)
    pooled = _sc_pool(idx2d, embed)
    return _tc_linear(pooled, W, b.reshape(1, _ODIM))


# speedup vs baseline: 2.6971x; 2.6971x over previous
import functools

import jax
import jax.numpy as jnp
from jax import lax
from jax.experimental import pallas as pl
from jax.experimental.pallas import tpu as pltpu
from jax.experimental.pallas import tpu_sc as plsc

_VOCAB = 1000000
_D = 64
_ODIM = 128
_B = 16384
_CTX = 50

_NC = 2
_NS = 16
_NW = _NC * _NS
_BPW = _B // _NW
_CB = 8
_NCHUNK = _BPW // _CB


def _sc_pool(idx2d, embed):
    mesh = plsc.VectorSubcoreMesh(core_axis_name="c", subcore_axis_name="s")

    @functools.partial(
        pl.kernel,
        mesh=mesh,
        compiler_params=pltpu.CompilerParams(use_tc_tiling_on_sc=False),
        out_type=jax.ShapeDtypeStruct((_B, _D), jnp.float32),
        scratch_types=[
            pltpu.VMEM((_BPW, _CTX), jnp.int32),
            pltpu.VMEM((2, _CB, _CTX, _D), jnp.float32),
            pltpu.VMEM((2, _CB, _D), jnp.float32),
            pltpu.SemaphoreType.DMA,
            pltpu.SemaphoreType.DMA,
            pltpu.SemaphoreType.DMA,
            pltpu.SemaphoreType.DMA,
        ],
    )
    def k(idx_hbm, table_hbm, out_hbm, idx_v, rows_v, acc_v, g0, g1, o0, o1):
        wid = lax.axis_index("s") * _NC + lax.axis_index("c")
        b0w = wid * _BPW
        gsem = (g0, g1)
        osem = (o0, o1)

        pltpu.sync_copy(idx_hbm.at[pl.ds(b0w, _BPW)], idx_v)

        def gather_descs(i, par):
            return [
                pltpu.make_async_copy(
                    table_hbm.at[idx_v.at[i * _CB + bb]],
                    rows_v.at[par, bb],
                    gsem[par],
                )
                for bb in range(_CB)
            ]

        def fire(i, par):
            for d in gather_descs(i, par):
                d.start()

        def drain(i, par):
            for d in gather_descs(i, par):
                d.wait()

        def pool(i, par):
            for bb in range(_CB):
                zeros = jnp.zeros((16,), jnp.float32)

                def ctx_body(c, acc, bb=bb, par=par):
                    r = c * 5
                    a0, a1, a2, a3 = acc
                    for u in range(5):
                        a0 = a0 + rows_v[par, bb, r + u, pl.ds(0, 16)]
                        a1 = a1 + rows_v[par, bb, r + u, pl.ds(16, 16)]
                        a2 = a2 + rows_v[par, bb, r + u, pl.ds(32, 16)]
                        a3 = a3 + rows_v[par, bb, r + u, pl.ds(48, 16)]
                    return (a0, a1, a2, a3)

                a0, a1, a2, a3 = lax.fori_loop(
                    0, _CTX // 5, ctx_body, (zeros, zeros, zeros, zeros))
                acc_v[par, bb, pl.ds(0, 16)] = a0
                acc_v[par, bb, pl.ds(16, 16)] = a1
                acc_v[par, bb, pl.ds(32, 16)] = a2
                acc_v[par, bb, pl.ds(48, 16)] = a3

        def out_desc(i, par):
            return pltpu.make_async_copy(
                acc_v.at[par],
                out_hbm.at[pl.ds(b0w + i * _CB, _CB)],
                osem[par],
            )

        fire(0, 0)

        def pair_body(p, carry):
            for q in range(2):
                i = 2 * p + q
                par = q
                drain(i, par)

                @pl.when(i + 1 < _NCHUNK)
                def _():
                    fire(i + 1, 1 - par)

                @pl.when(i >= 2)
                def _():
                    out_desc(i - 2, par).wait()

                pool(i, par)
                out_desc(i, par).start()
            return carry

        lax.fori_loop(0, _NCHUNK // 2, pair_body, 0)

        out_desc(_NCHUNK - 2, 0).wait()
        out_desc(_NCHUNK - 1, 1).wait()

    return k(idx2d, embed)


def _tc_linear(pooled, W, b2d):
    BB = 2048

    def body(x_ref, w_ref, b_ref, o_ref):
        o_ref[...] = lax.dot_general(
            x_ref[...], w_ref[...], (((1,), (1,)), ((), ())),
            preferred_element_type=jnp.float32,
        ) + b_ref[...]

    return pl.pallas_call(
        body,
        grid=(_B // BB,),
        in_specs=[
            pl.BlockSpec((BB, _D), lambda i: (i, 0)),
            pl.BlockSpec((_ODIM, _D), lambda i: (0, 0)),
            pl.BlockSpec((1, _ODIM), lambda i: (0, 0)),
        ],
        out_specs=pl.BlockSpec((BB, _ODIM), lambda i: (i, 0)),
        out_shape=jax.ShapeDtypeStruct((_B, _ODIM), jnp.float32),
    )(pooled, W, b2d)


def kernel(inputs, embed, W, b):
    pooled = _sc_pool(inputs.astype(jnp.int32), embed)
    return _tc_linear(pooled, W, b.reshape(1, _ODIM))
